# Initial kernel scaffold; baseline (speedup 1.0000x reference)
import jax
import jax.numpy as jnp
from jax.experimental import pallas as pl


def kernel(hidden_states, gate_weight, expert_gate, expert_up, expert_down, shared_gate, shared_up, shared_down):
    raise NotImplementedError("write your pallas kernel here")


# speedup vs baseline: 3.3901x; 3.3901x over previous
import jax
import jax.numpy as jnp
from jax.experimental import pallas as pl
from jax.experimental.pallas import tpu as pltpu

S = 2048
H = 2048
NE = 64
TOPK = 2
DFF = 512
SFF = 1024

TM = 64
PAIRS = S * TOPK
NT = PAIRS // TM + NE
PADDED = NT * TM

TOK_TILE = 256
_VMEM = pltpu.CompilerParams(vmem_limit_bytes=100 * 1024 * 1024)


def _gate_kernel(x_ref, gw_ref, w_ref, i_ref):
    x = x_ref[...]
    logits = jax.lax.dot_general(
        x, gw_ref[...], (((1,), (1,)), ((), ())),
        preferred_element_type=jnp.float32)
    m = jnp.max(logits, axis=1, keepdims=True)
    e = jnp.exp(logits - m)
    p = e / jnp.sum(e, axis=1, keepdims=True)
    ii = jax.lax.broadcasted_iota(jnp.int32, p.shape, 1)
    m1 = jnp.max(p, axis=1, keepdims=True)
    i1 = jnp.min(jnp.where(p == m1, ii, NE), axis=1, keepdims=True)
    p2 = jnp.where(ii == i1, -1.0, p)
    m2 = jnp.max(p2, axis=1, keepdims=True)
    i2 = jnp.min(jnp.where(p2 == m2, ii, NE), axis=1, keepdims=True)
    denom = m1 + m2 + 1e-20
    w_ref[...] = jnp.concatenate([m1 / denom, m2 / denom], axis=1)
    i_ref[...] = jnp.concatenate([i1, i2], axis=1)


def _moe_kernel(te_ref, st_ref, sw_ref, x_ref, g_ref, u_ref, d_ref,
                out_ref, xs, ys):
    i = pl.program_id(0)

    @pl.when(i == 0)
    def _():
        out_ref[...] = jnp.zeros_like(out_ref)

    base = i * TM
    for j in range(TM):
        xs[j, :] = x_ref[st_ref[base + j], :]
    xv = xs[...]
    a = jax.lax.dot_general(xv, g_ref[0], (((1,), (1,)), ((), ())),
                            preferred_element_type=jnp.float32)
    b = jax.lax.dot_general(xv, u_ref[0], (((1,), (1,)), ((), ())),
                            preferred_element_type=jnp.float32)
    h1 = (a * jax.nn.sigmoid(a)) * b
    ys[...] = jax.lax.dot_general(h1, d_ref[0], (((1,), (1,)), ((), ())),
                                  preferred_element_type=jnp.float32)
    for j in range(TM):
        tok = st_ref[base + j]
        out_ref[tok, :] = out_ref[tok, :] + sw_ref[base + j] * ys[j, :]


def _shared_kernel(x_ref, r_ref, sg_ref, su_ref, sd_ref, o_ref):
    x = x_ref[...]
    a = jax.lax.dot_general(x, sg_ref[...], (((1,), (1,)), ((), ())),
                            preferred_element_type=jnp.float32)
    b = jax.lax.dot_general(x, su_ref[...], (((1,), (1,)), ((), ())),
                            preferred_element_type=jnp.float32)
    h1 = (a * jax.nn.sigmoid(a)) * b
    o_ref[...] = r_ref[...] + jax.lax.dot_general(
        h1, sd_ref[...], (((1,), (1,)), ((), ())),
        preferred_element_type=jnp.float32)


def kernel(hidden_states, gate_weight, expert_gate, expert_up, expert_down,
           shared_gate, shared_up, shared_down):
    b, s, h = hidden_states.shape
    x = hidden_states.reshape(s, h)

    n_tok_tiles = s // TOK_TILE
    topk_w, topk_idx = pl.pallas_call(
        _gate_kernel,
        grid=(n_tok_tiles,),
        in_specs=[
            pl.BlockSpec((TOK_TILE, H), lambda i: (i, 0)),
            pl.BlockSpec((NE, H), lambda i: (0, 0)),
        ],
        out_specs=[
            pl.BlockSpec((TOK_TILE, TOPK), lambda i: (i, 0)),
            pl.BlockSpec((TOK_TILE, TOPK), lambda i: (i, 0)),
        ],
        out_shape=[
            jax.ShapeDtypeStruct((S, TOPK), jnp.float32),
            jax.ShapeDtypeStruct((S, TOPK), jnp.int32),
        ],
    )(x, gate_weight)

    e_flat = topk_idx.reshape(-1)
    w_flat = topk_w.reshape(-1)
    order = jnp.argsort(e_flat)
    sorted_e = e_flat[order]
    counts = jnp.bincount(e_flat, length=NE)
    raw_off = jnp.concatenate(
        [jnp.zeros((1,), jnp.int32), jnp.cumsum(counts)[:-1].astype(jnp.int32)])
    pad_counts = ((counts + TM - 1) // TM) * TM
    pad_off = jnp.concatenate(
        [jnp.zeros((1,), jnp.int32),
         jnp.cumsum(pad_counts)[:-1].astype(jnp.int32)])
    r = jnp.arange(PAIRS, dtype=jnp.int32)
    slot = pad_off[sorted_e] + (r - raw_off[sorted_e])
    slot_token = jnp.zeros((PADDED,), jnp.int32).at[slot].set(
        (order // TOPK).astype(jnp.int32))
    slot_w = jnp.zeros((PADDED,), jnp.float32).at[slot].set(w_flat[order])
    tile_expert = jnp.clip(
        jnp.searchsorted(pad_off, jnp.arange(NT, dtype=jnp.int32) * TM,
                         side='right') - 1, 0, NE - 1).astype(jnp.int32)

    routed = pl.pallas_call(
        _moe_kernel,
        grid_spec=pltpu.PrefetchScalarGridSpec(
            num_scalar_prefetch=3,
            grid=(NT,),
            in_specs=[
                pl.BlockSpec((S, H), lambda i, te, st, sw: (0, 0)),
                pl.BlockSpec((1, DFF, H), lambda i, te, st, sw: (te[i], 0, 0)),
                pl.BlockSpec((1, DFF, H), lambda i, te, st, sw: (te[i], 0, 0)),
                pl.BlockSpec((1, H, DFF), lambda i, te, st, sw: (te[i], 0, 0)),
            ],
            out_specs=pl.BlockSpec((S, H), lambda i, te, st, sw: (0, 0)),
            scratch_shapes=[
                pltpu.VMEM((TM, H), jnp.float32),
                pltpu.VMEM((TM, H), jnp.float32),
            ],
        ),
        out_shape=jax.ShapeDtypeStruct((S, H), jnp.float32),
        compiler_params=_VMEM,
    )(tile_expert, slot_token, slot_w, x, expert_gate, expert_up, expert_down)

    y = pl.pallas_call(
        _shared_kernel,
        grid=(n_tok_tiles,),
        in_specs=[
            pl.BlockSpec((TOK_TILE, H), lambda i: (i, 0)),
            pl.BlockSpec((TOK_TILE, H), lambda i: (i, 0)),
            pl.BlockSpec((SFF, H), lambda i: (0, 0)),
            pl.BlockSpec((SFF, H), lambda i: (0, 0)),
            pl.BlockSpec((H, SFF), lambda i: (0, 0)),
        ],
        out_specs=pl.BlockSpec((TOK_TILE, H), lambda i: (i, 0)),
        out_shape=jax.ShapeDtypeStruct((S, H), jnp.float32),
        compiler_params=_VMEM,
    )(x, routed, shared_gate, shared_up, shared_down)

    return y.reshape(b, s, h)


# speedup vs baseline: 4.7737x; 1.4081x over previous
import jax
import jax.numpy as jnp
from jax import lax
from jax.experimental import pallas as pl
from jax.experimental.pallas import tpu as pltpu
from jax.experimental.pallas import tpu_sc as plsc

S = 2048
H = 2048
NE = 64
TOPK = 2
DFF = 512
SFF = 1024

TM = 64
PAIRS = S * TOPK
NT = PAIRS // TM + NE
PADDED = NT * TM
_LANES = 16

TOK_TILE = 256
_VMEM = pltpu.CompilerParams(vmem_limit_bytes=100 * 1024 * 1024)


def _gate_kernel(x_ref, gw_ref, w_ref, i_ref):
    x = x_ref[...]
    logits = jax.lax.dot_general(
        x, gw_ref[...], (((1,), (1,)), ((), ())),
        preferred_element_type=jnp.float32)
    m = jnp.max(logits, axis=1, keepdims=True)
    e = jnp.exp(logits - m)
    p = e / jnp.sum(e, axis=1, keepdims=True)
    ii = jax.lax.broadcasted_iota(jnp.int32, p.shape, 1)
    m1 = jnp.max(p, axis=1, keepdims=True)
    i1 = jnp.min(jnp.where(p == m1, ii, NE), axis=1, keepdims=True)
    p2 = jnp.where(ii == i1, -1.0, p)
    m2 = jnp.max(p2, axis=1, keepdims=True)
    i2 = jnp.min(jnp.where(p2 == m2, ii, NE), axis=1, keepdims=True)
    denom = m1 + m2 + 1e-20
    w_ref[...] = jnp.concatenate([m1 / denom, m2 / denom], axis=1)
    i_ref[...] = jnp.concatenate([i1, i2], axis=1)


def _sc_at(ref, i):
    return ref[pl.ds(i, _LANES)][0]


def _layout_sc(tok_hbm, w_hbm, cnt_hbm, roff_hbm, poff_hbm,
               st_out, sw_out,
               tsv, wsv, stv, swv, cntv, roffv, poffv):
    c = lax.axis_index("c")
    s_ = lax.axis_index("s")

    @pl.when(jnp.logical_and(c == 0, s_ == 0))
    def _():
        pltpu.sync_copy(tok_hbm, tsv.at[pl.ds(0, PAIRS)])
        pltpu.sync_copy(w_hbm, wsv.at[pl.ds(0, PAIRS)])
        pltpu.sync_copy(cnt_hbm, cntv.at[pl.ds(0, NE)])
        pltpu.sync_copy(roff_hbm, roffv.at[pl.ds(0, NE)])
        pltpu.sync_copy(poff_hbm, poffv.at[pl.ds(0, NE)])

        def per_expert(e, _):
            ce = _sc_at(cntv, e)
            s0 = _sc_at(roffv, e)
            d0 = _sc_at(poffv, e)
            nfull = ce // _LANES

            def cp(k, _2):
                o = k * _LANES
                stv[pl.ds(d0 + o, _LANES)] = tsv[pl.ds(s0 + o, _LANES)]
                swv[pl.ds(d0 + o, _LANES)] = wsv[pl.ds(s0 + o, _LANES)]
                return 0
            lax.fori_loop(0, nfull, cp, 0)

            rem = ce - nfull * _LANES

            @pl.when(jnp.logical_and(rem > 0, ce >= _LANES))
            def _t1():
                o = ce - _LANES
                stv[pl.ds(d0 + o, _LANES)] = tsv[pl.ds(s0 + o, _LANES)]
                swv[pl.ds(d0 + o, _LANES)] = wsv[pl.ds(s0 + o, _LANES)]

            @pl.when(jnp.logical_and(rem > 0, ce < _LANES))
            def _t2():
                stv[pl.ds(d0, _LANES)] = tsv[pl.ds(s0, _LANES)]
                swv[pl.ds(d0, _LANES)] = wsv[pl.ds(s0, _LANES)]
            return 0

        lax.fori_loop(0, NE, per_expert, 0)

        pltpu.sync_copy(stv.at[pl.ds(0, PADDED)], st_out)
        pltpu.sync_copy(swv.at[pl.ds(0, PADDED)], sw_out)


def _moe_kernel(nu_ref, te_ref, se_ref, st_ref, sw_ref,
                x_ref, g_ref, u_ref, d_ref, out_ref, xs, ys):
    i = pl.program_id(0)

    @pl.when(i == 0)
    def _():
        out_ref[...] = jnp.zeros_like(out_ref)

    @pl.when(i < nu_ref[0])
    def _():
        base = i * TM
        send = se_ref[i]
        for j in range(TM):
            tok = jnp.clip(st_ref[base + j], 0, S - 1)
            xs[j, :] = x_ref[tok, :]
        xv = xs[...]
        a = jax.lax.dot_general(xv, g_ref[0], (((1,), (1,)), ((), ())),
                                preferred_element_type=jnp.float32)
        b = jax.lax.dot_general(xv, u_ref[0], (((1,), (1,)), ((), ())),
                                preferred_element_type=jnp.float32)
        h1 = (a * jax.nn.sigmoid(a)) * b
        ys[...] = jax.lax.dot_general(h1, d_ref[0], (((1,), (1,)), ((), ())),
                                      preferred_element_type=jnp.float32)
        for j in range(TM):
            tok = jnp.clip(st_ref[base + j], 0, S - 1)
            w = jnp.where(base + j < send, sw_ref[base + j], 0.0)
            out_ref[tok, :] = out_ref[tok, :] + w * ys[j, :]


def _shared_kernel(x_ref, r_ref, sg_ref, su_ref, sd_ref, o_ref):
    x = x_ref[...]
    a = jax.lax.dot_general(x, sg_ref[...], (((1,), (1,)), ((), ())),
                            preferred_element_type=jnp.float32)
    b = jax.lax.dot_general(x, su_ref[...], (((1,), (1,)), ((), ())),
                            preferred_element_type=jnp.float32)
    h1 = (a * jax.nn.sigmoid(a)) * b
    o_ref[...] = r_ref[...] + jax.lax.dot_general(
        h1, sd_ref[...], (((1,), (1,)), ((), ())),
        preferred_element_type=jnp.float32)


def kernel(hidden_states, gate_weight, expert_gate, expert_up, expert_down,
           shared_gate, shared_up, shared_down):
    b, s, h = hidden_states.shape
    x = hidden_states.reshape(s, h)

    n_tok_tiles = s // TOK_TILE
    topk_w, topk_idx = pl.pallas_call(
        _gate_kernel,
        grid=(n_tok_tiles,),
        in_specs=[
            pl.BlockSpec((TOK_TILE, H), lambda i: (i, 0)),
            pl.BlockSpec((NE, H), lambda i: (0, 0)),
        ],
        out_specs=[
            pl.BlockSpec((TOK_TILE, TOPK), lambda i: (i, 0)),
            pl.BlockSpec((TOK_TILE, TOPK), lambda i: (i, 0)),
        ],
        out_shape=[
            jax.ShapeDtypeStruct((S, TOPK), jnp.float32),
            jax.ShapeDtypeStruct((S, TOPK), jnp.int32),
        ],
    )(x, gate_weight)

    e_flat = topk_idx.reshape(-1)
    w_flat = topk_w.reshape(-1)
    order = jnp.argsort(e_flat)
    tok_sorted = (order // TOPK).astype(jnp.int32)
    w_sorted = w_flat[order]
    counts = jnp.bincount(e_flat, length=NE).astype(jnp.int32)
    raw_off = jnp.concatenate(
        [jnp.zeros((1,), jnp.int32), jnp.cumsum(counts)[:-1].astype(jnp.int32)])
    pad_counts = ((counts + TM - 1) // TM) * TM
    pad_off = jnp.concatenate(
        [jnp.zeros((1,), jnp.int32),
         jnp.cumsum(pad_counts)[:-1].astype(jnp.int32)])
    n_used = (jnp.sum(pad_counts) // TM).astype(jnp.int32).reshape(1)
    tile_expert = jnp.clip(
        jnp.searchsorted(pad_off, jnp.arange(NT, dtype=jnp.int32) * TM,
                         side='right') - 1, 0, NE - 1).astype(jnp.int32)
    seg_end = pad_off[tile_expert] + counts[tile_expert]
    last_e = tile_expert[jnp.maximum(n_used[0] - 1, 0)]
    tile_idx = jnp.arange(NT, dtype=jnp.int32)
    tile_expert = jnp.where(tile_idx < n_used[0], tile_expert, last_e)

    slot_token, slot_w = pl.kernel(
        _layout_sc,
        out_type=[
            jax.ShapeDtypeStruct((PADDED,), jnp.int32),
            jax.ShapeDtypeStruct((PADDED,), jnp.float32),
        ],
        mesh=plsc.VectorSubcoreMesh(core_axis_name="c", subcore_axis_name="s",
                                    num_cores=2, num_subcores=16),
        scratch_types=[
            pltpu.VMEM((PAIRS + 128,), jnp.int32),
            pltpu.VMEM((PAIRS + 128,), jnp.float32),
            pltpu.VMEM((PADDED + 128,), jnp.int32),
            pltpu.VMEM((PADDED + 128,), jnp.float32),
            pltpu.VMEM((128,), jnp.int32),
            pltpu.VMEM((128,), jnp.int32),
            pltpu.VMEM((128,), jnp.int32),
        ],
    )(tok_sorted, w_sorted, counts, raw_off, pad_off)

    routed = pl.pallas_call(
        _moe_kernel,
        grid_spec=pltpu.PrefetchScalarGridSpec(
            num_scalar_prefetch=5,
            grid=(NT,),
            in_specs=[
                pl.BlockSpec((S, H), lambda i, nu, te, se, st, sw: (0, 0)),
                pl.BlockSpec((1, DFF, H),
                             lambda i, nu, te, se, st, sw: (te[i], 0, 0)),
                pl.BlockSpec((1, DFF, H),
                             lambda i, nu, te, se, st, sw: (te[i], 0, 0)),
                pl.BlockSpec((1, H, DFF),
                             lambda i, nu, te, se, st, sw: (te[i], 0, 0)),
            ],
            out_specs=pl.BlockSpec((S, H),
                                   lambda i, nu, te, se, st, sw: (0, 0)),
            scratch_shapes=[
                pltpu.VMEM((TM, H), jnp.float32),
                pltpu.VMEM((TM, H), jnp.float32),
            ],
        ),
        out_shape=jax.ShapeDtypeStruct((S, H), jnp.float32),
        compiler_params=_VMEM,
    )(n_used, tile_expert, seg_end, slot_token, slot_w, x,
      expert_gate, expert_up, expert_down)

    y = pl.pallas_call(
        _shared_kernel,
        grid=(n_tok_tiles,),
        in_specs=[
            pl.BlockSpec((TOK_TILE, H), lambda i: (i, 0)),
            pl.BlockSpec((TOK_TILE, H), lambda i: (i, 0)),
            pl.BlockSpec((SFF, H), lambda i: (0, 0)),
            pl.BlockSpec((SFF, H), lambda i: (0, 0)),
            pl.BlockSpec((H, SFF), lambda i: (0, 0)),
        ],
        out_specs=pl.BlockSpec((TOK_TILE, H), lambda i: (i, 0)),
        out_shape=jax.ShapeDtypeStruct((S, H), jnp.float32),
        compiler_params=_VMEM,
    )(x, routed, shared_gate, shared_up, shared_down)

    return y.reshape(b, s, h)
